# baseline (speedup 1.0000x reference)
import numpy as np
import jax
import jax.numpy as jnp
from jax.experimental import pallas as pl
from jax.experimental.pallas import tpu as pltpu

B, S, HM = 1, 2048, 1024
NH, HD = 16, 64
RATIO, TOPK, WINDOW = 4, 8, 128
ROPE_DIM = HD // 2
HALF = ROPE_DIM // 2
C = S // RATIO
TS = 256
CB = TS // RATIO
KW = TS + WINDOW
NB = S // TS
SCALE = 1.0 / np.sqrt(HD)
NEG = -1e9


def _rope_tables(pos_f):
    j = jax.lax.broadcasted_iota(jnp.int32, pos_f.shape, 1).astype(jnp.float32)
    freqs = jnp.exp(j * (-np.log(10000.0) / HALF))
    ang = pos_f * freqs
    return jnp.cos(ang), jnp.sin(ang)


def _norm_rope(x, g, cos, sin):
    ms = jnp.mean(x * x, axis=-1, keepdims=True)
    xn = x * jax.lax.rsqrt(ms + 1e-6) * g
    x1 = xn[:, :HALF]
    x2 = xn[:, HALF:ROPE_DIM]
    xp = xn[:, ROPE_DIM:]
    r1 = x1 * cos - x2 * sin
    r2 = x1 * sin + x2 * cos
    return jnp.concatenate([r1, r2, xp], axis=1)


def _proj_kernel(hs_ref, Wq_ref, bq_ref, Wcomp_ref, bcomp_ref, Wk_ref, bk_ref,
                 Wv_ref, bv_ref, Wlk_ref, blk_ref, Wlv_ref, blv_ref,
                 gq_ref, gk_ref,
                 q_ref, lk_ref, lv_ref, ck_ref, cv_ref):
    i = pl.program_id(0)
    hs = hs_ref[:]

    pos = (i * TS + jax.lax.broadcasted_iota(jnp.int32, (TS, HALF), 0)
           ).astype(jnp.float32)
    cos, sin = _rope_tables(pos)

    q = jnp.dot(hs, Wq_ref[:], preferred_element_type=jnp.float32) + bq_ref[:]
    lk = jnp.dot(hs, Wlk_ref[:], preferred_element_type=jnp.float32) + blk_ref[:]
    lv_ref[:] = jnp.dot(hs, Wlv_ref[:], preferred_element_type=jnp.float32) + blv_ref[:]
    gq = gq_ref[:]
    gk = gk_ref[:]
    for h in range(NH):
        sl = slice(h * HD, (h + 1) * HD)
        q_ref[:, sl] = _norm_rope(q[:, sl], gq, cos, sin)
        lk_ref[:, sl] = _norm_rope(lk[:, sl], gk, cos, sin)

    HIP = jax.lax.Precision.HIGHEST
    bf = lambda x: x.astype(jnp.bfloat16).astype(jnp.float32)
    sc = jnp.dot(hs, Wcomp_ref[:], preferred_element_type=jnp.float32) + bcomp_ref[:]
    e = jnp.exp(sc - jnp.max(sc))
    arow = jax.lax.broadcasted_iota(jnp.int32, (CB, TS), 0)
    acol = jax.lax.broadcasted_iota(jnp.int32, (CB, TS), 1)
    A = (acol // RATIO == arow).astype(jnp.float32)
    gs = jnp.dot(A, e, preferred_element_type=jnp.float32, precision=HIP)
    trow = jax.lax.broadcasted_iota(jnp.int32, (TS, CB), 0)
    tcol = jax.lax.broadcasted_iota(jnp.int32, (TS, CB), 1)
    At = (trow // RATIO == tcol).astype(jnp.float32)
    gsum_t = jnp.dot(At, gs, preferred_element_type=jnp.float32, precision=HIP)
    w = e / gsum_t
    entries = jnp.dot(A, bf(hs) * bf(w), preferred_element_type=jnp.float32,
                      precision=HIP)

    ck = jnp.dot(entries, Wk_ref[:], preferred_element_type=jnp.float32) + bk_ref[:]
    cv = jnp.dot(entries, Wv_ref[:], preferred_element_type=jnp.float32) + bv_ref[:]
    cpos = ((i * CB + jax.lax.broadcasted_iota(jnp.int32, (CB, HALF), 0))
            * RATIO + (RATIO - 1)).astype(jnp.float32)
    cosc, sinc = _rope_tables(cpos)
    ck_ref[:] = _norm_rope(ck, gk, cosc, sinc)
    cv_ref[:] = cv


def _attn_kernel(q_ref, lk_ref, lv_ref, ck_ref, cv_ref, Wo_ref, bo_ref, o_ref,
                 m_scr):
    i = pl.program_id(0)

    ks = pl.multiple_of(jnp.maximum(i * TS - WINDOW, 0), WINDOW)
    kwin = lk_ref[pl.ds(ks, KW), :]
    vwin = lv_ref[pl.ds(ks, KW), :]
    qp = i * TS + jax.lax.broadcasted_iota(jnp.int32, (TS, KW), 0)
    kp = ks + jax.lax.broadcasted_iota(jnp.int32, (TS, KW), 1)
    diff = qp - kp
    vis_l = (diff >= 0) & (diff <= WINDOW)

    def body(CW):
        ck = ck_ref[:CW, :]
        cv = cv_ref[:CW, :]
        qpos_c = i * TS + jax.lax.broadcasted_iota(jnp.int32, (TS, CW), 0)
        cstart = RATIO * jax.lax.broadcasted_iota(jnp.int32, (TS, CW), 1)
        vis_c = cstart <= qpos_c

        for h in range(NH):
            hs_ = slice(h * HD, (h + 1) * HD)
            qb = q_ref[:, hs_]

            s = jax.lax.dot_general(qb, ck, (((1,), (1,)), ((), ())),
                                    preferred_element_type=jnp.float32) * SCALE
            s = jnp.where(vis_c, s, NEG)

            work = s
            m0 = None
            for _ in range(TOPK):
                mk = jnp.max(work, axis=1, keepdims=True)
                if m0 is None:
                    m0 = mk
                thresh = mk
                work = jnp.where(work >= mk, -jnp.inf, work)
            P = jnp.where(s >= thresh, jnp.exp(s - m0), 0.0)
            norm = jnp.sum(P, axis=1, keepdims=True)
            ctx_s = jnp.dot(P, cv, preferred_element_type=jnp.float32) / norm

            sl = jax.lax.dot_general(qb, kwin[:, hs_], (((1,), (1,)), ((), ())),
                                     preferred_element_type=jnp.float32) * SCALE
            sl = jnp.where(vis_l, sl, NEG)
            lm = jnp.max(sl, axis=1, keepdims=True)
            le = jnp.exp(sl - lm)
            ctx_l = (jnp.dot(le, vwin[:, hs_], preferred_element_type=jnp.float32)
                     / jnp.sum(le, axis=1, keepdims=True))

            m_scr[:, hs_] = (ctx_s + ctx_l) * 0.5
        o_ref[:] = (jnp.dot(m_scr[:], Wo_ref[:],
                            preferred_element_type=jnp.float32) + bo_ref[:])

    @pl.when(i < NB // 2)
    def _():
        body(C // 2)

    @pl.when(i >= NB // 2)
    def _():
        body(C)


def kernel(hidden_states, Wq, bq, Wcomp, bcomp, Wk, bk, Wv, bv,
           Wlk, blk, Wlv, blv, gq, gk, Wo, bo):
    hs = hidden_states.reshape(S, HM)
    bq2 = bq.reshape(1, -1)
    bcomp2 = bcomp.reshape(1, 1)
    bk2 = bk.reshape(1, -1)
    bv2 = bv.reshape(1, -1)
    blk2 = blk.reshape(1, -1)
    blv2 = blv.reshape(1, -1)
    gq2 = gq.reshape(1, -1)
    gk2 = gk.reshape(1, -1)
    bo2 = bo.reshape(1, -1)

    full = lambda shape: pl.BlockSpec(shape, lambda i: (0, 0))
    q, lk, lv, ck, cv = pl.pallas_call(
        _proj_kernel,
        grid=(NB,),
        in_specs=[
            pl.BlockSpec((TS, HM), lambda i: (i, 0)),
            full((HM, NH * HD)), full((1, NH * HD)),
            full((HM, 1)), full((1, 1)),
            full((HM, HD)), full((1, HD)),
            full((HM, HD)), full((1, HD)),
            full((HM, NH * HD)), full((1, NH * HD)),
            full((HM, NH * HD)), full((1, NH * HD)),
            full((1, HD)), full((1, HD)),
        ],
        out_specs=(
            pl.BlockSpec((TS, NH * HD), lambda i: (i, 0)),
            pl.BlockSpec((TS, NH * HD), lambda i: (i, 0)),
            pl.BlockSpec((TS, NH * HD), lambda i: (i, 0)),
            pl.BlockSpec((CB, HD), lambda i: (i, 0)),
            pl.BlockSpec((CB, HD), lambda i: (i, 0)),
        ),
        out_shape=(
            jax.ShapeDtypeStruct((S, NH * HD), jnp.float32),
            jax.ShapeDtypeStruct((S, NH * HD), jnp.float32),
            jax.ShapeDtypeStruct((S, NH * HD), jnp.float32),
            jax.ShapeDtypeStruct((C, HD), jnp.float32),
            jax.ShapeDtypeStruct((C, HD), jnp.float32),
        ),
    )(hs, Wq, bq2, Wcomp, bcomp2, Wk, bk2, Wv, bv2, Wlk, blk2, Wlv, blv2,
      gq2, gk2)

    out = pl.pallas_call(
        _attn_kernel,
        grid=(NB,),
        in_specs=[
            pl.BlockSpec((TS, NH * HD), lambda i: (i, 0)),
            pl.BlockSpec((S, NH * HD), lambda i: (0, 0)),
            pl.BlockSpec((S, NH * HD), lambda i: (0, 0)),
            pl.BlockSpec((C, HD), lambda i: (0, 0)),
            pl.BlockSpec((C, HD), lambda i: (0, 0)),
            pl.BlockSpec((NH * HD, HM), lambda i: (0, 0)),
            pl.BlockSpec((1, HM), lambda i: (0, 0)),
        ],
        out_specs=pl.BlockSpec((TS, HM), lambda i: (i, 0)),
        out_shape=jax.ShapeDtypeStruct((S, HM), jnp.float32),
        scratch_shapes=[pltpu.VMEM((TS, NH * HD), jnp.float32)],
    )(q, lk, lv, ck, cv, Wo, bo2)

    return out.reshape(B, S, HM)


# speedup vs baseline: 1.5411x; 1.5411x over previous
import numpy as np
import jax
import jax.numpy as jnp
from jax.experimental import pallas as pl

B, S, HM = 1, 2048, 1024
NH, HD = 16, 64
RATIO, TOPK, WINDOW = 4, 8, 128
ROPE_DIM = HD // 2
HALF = ROPE_DIM // 2
C = S // RATIO
TS = 256
CB = TS // RATIO
KW = TS + WINDOW
NB = S // TS
SCALE = 1.0 / np.sqrt(HD)
NEG = -1e9


def _rope_tables(pos_f):
    j = jax.lax.broadcasted_iota(jnp.int32, pos_f.shape, 1).astype(jnp.float32)
    freqs = jnp.exp(j * (-np.log(10000.0) / HALF))
    ang = pos_f * freqs
    return jnp.cos(ang), jnp.sin(ang)


def _norm_rope(x, g, cos, sin):
    ms = jnp.mean(x * x, axis=-1, keepdims=True)
    xn = x * jax.lax.rsqrt(ms + 1e-6) * g
    x1 = xn[:, :HALF]
    x2 = xn[:, HALF:ROPE_DIM]
    xp = xn[:, ROPE_DIM:]
    r1 = x1 * cos - x2 * sin
    r2 = x1 * sin + x2 * cos
    return jnp.concatenate([r1, r2, xp], axis=1)


def _proj_kernel(hs_ref, Wq_ref, bq_ref, Wcomp_ref, bcomp_ref, Wk_ref, bk_ref,
                 Wv_ref, bv_ref, Wlk_ref, blk_ref, Wlv_ref, blv_ref,
                 gq_ref, gk_ref,
                 q_ref, lk_ref, lv_ref, ck_ref, cv_ref):
    i = pl.program_id(0)
    hs = hs_ref[:]

    pos = (i * TS + jax.lax.broadcasted_iota(jnp.int32, (TS, HALF), 0)
           ).astype(jnp.float32)
    cos, sin = _rope_tables(pos)

    q = jnp.dot(hs, Wq_ref[:], preferred_element_type=jnp.float32) + bq_ref[:]
    lk = jnp.dot(hs, Wlk_ref[:], preferred_element_type=jnp.float32) + blk_ref[:]
    lv_ref[:] = jnp.dot(hs, Wlv_ref[:], preferred_element_type=jnp.float32) + blv_ref[:]
    gq = gq_ref[:]
    gk = gk_ref[:]
    for h in range(NH):
        sl = slice(h * HD, (h + 1) * HD)
        q_ref[:, sl] = _norm_rope(q[:, sl], gq, cos, sin)
        lk_ref[:, sl] = _norm_rope(lk[:, sl], gk, cos, sin)

    HIP = jax.lax.Precision.HIGHEST
    bf = lambda x: x.astype(jnp.bfloat16).astype(jnp.float32)
    sc = jnp.dot(hs, Wcomp_ref[:], preferred_element_type=jnp.float32) + bcomp_ref[:]
    e = jnp.exp(sc - jnp.max(sc))
    arow = jax.lax.broadcasted_iota(jnp.int32, (CB, TS), 0)
    acol = jax.lax.broadcasted_iota(jnp.int32, (CB, TS), 1)
    A = (acol // RATIO == arow).astype(jnp.float32)
    gs = jnp.dot(A, e, preferred_element_type=jnp.float32, precision=HIP)
    trow = jax.lax.broadcasted_iota(jnp.int32, (TS, CB), 0)
    tcol = jax.lax.broadcasted_iota(jnp.int32, (TS, CB), 1)
    At = (trow // RATIO == tcol).astype(jnp.float32)
    gsum_t = jnp.dot(At, gs, preferred_element_type=jnp.float32, precision=HIP)
    w = e / gsum_t
    entries = jnp.dot(A, bf(hs) * bf(w), preferred_element_type=jnp.float32,
                      precision=HIP)

    ck = jnp.dot(entries, Wk_ref[:], preferred_element_type=jnp.float32) + bk_ref[:]
    cv = jnp.dot(entries, Wv_ref[:], preferred_element_type=jnp.float32) + bv_ref[:]
    cpos = ((i * CB + jax.lax.broadcasted_iota(jnp.int32, (CB, HALF), 0))
            * RATIO + (RATIO - 1)).astype(jnp.float32)
    cosc, sinc = _rope_tables(cpos)
    ck_ref[:] = _norm_rope(ck, gk, cosc, sinc)
    cv_ref[:] = cv


def _attn_kernel(q_ref, lk_ref, lv_ref, ck_ref, cv_ref, o_ref):
    i = pl.program_id(0)

    ks = pl.multiple_of(jnp.maximum(i * TS - WINDOW, 0), WINDOW)
    kwin = lk_ref[pl.ds(ks, KW), :]
    vwin = lv_ref[pl.ds(ks, KW), :]
    qp = i * TS + jax.lax.broadcasted_iota(jnp.int32, (TS, KW), 0)
    kp = ks + jax.lax.broadcasted_iota(jnp.int32, (TS, KW), 1)
    diff = qp - kp
    vis_l = (diff >= 0) & (diff <= WINDOW)

    def body(CW):
        ck = ck_ref[:CW, :]
        cv = cv_ref[:CW, :]
        qpos_c = i * TS + jax.lax.broadcasted_iota(jnp.int32, (TS, CW), 0)
        cstart = RATIO * jax.lax.broadcasted_iota(jnp.int32, (TS, CW), 1)
        vis_c = cstart <= qpos_c

        for h in range(NH):
            hs_ = slice(h * HD, (h + 1) * HD)
            qb = q_ref[:, hs_]

            s = jax.lax.dot_general(qb, ck, (((1,), (1,)), ((), ())),
                                    preferred_element_type=jnp.float32) * SCALE
            s = jnp.where(vis_c, s, NEG)

            work = s
            m0 = None
            for _ in range(TOPK):
                mk = jnp.max(work, axis=1, keepdims=True)
                if m0 is None:
                    m0 = mk
                thresh = mk
                work = jnp.where(work >= mk, -jnp.inf, work)
            P = jnp.where(s >= thresh, jnp.exp(s - m0), 0.0)
            norm = jnp.sum(P, axis=1, keepdims=True)
            ctx_s = jnp.dot(P, cv, preferred_element_type=jnp.float32) / norm

            sl = jax.lax.dot_general(qb, kwin[:, hs_], (((1,), (1,)), ((), ())),
                                     preferred_element_type=jnp.float32) * SCALE
            sl = jnp.where(vis_l, sl, NEG)
            lm = jnp.max(sl, axis=1, keepdims=True)
            le = jnp.exp(sl - lm)
            ctx_l = (jnp.dot(le, vwin[:, hs_], preferred_element_type=jnp.float32)
                     / jnp.sum(le, axis=1, keepdims=True))

            o_ref[:, hs_] = (ctx_s + ctx_l) * 0.5

    @pl.when(i < NB // 2)
    def _():
        body(C // 2)

    @pl.when(i >= NB // 2)
    def _():
        body(C)


def _out_kernel(m_ref, Wo_ref, bo_ref, o_ref):
    o_ref[:] = (jnp.dot(m_ref[:], Wo_ref[:], preferred_element_type=jnp.float32)
                + bo_ref[:])


def kernel(hidden_states, Wq, bq, Wcomp, bcomp, Wk, bk, Wv, bv,
           Wlk, blk, Wlv, blv, gq, gk, Wo, bo):
    hs = hidden_states.reshape(S, HM)
    bq2 = bq.reshape(1, -1)
    bcomp2 = bcomp.reshape(1, 1)
    bk2 = bk.reshape(1, -1)
    bv2 = bv.reshape(1, -1)
    blk2 = blk.reshape(1, -1)
    blv2 = blv.reshape(1, -1)
    gq2 = gq.reshape(1, -1)
    gk2 = gk.reshape(1, -1)
    bo2 = bo.reshape(1, -1)

    full = lambda shape: pl.BlockSpec(shape, lambda i: (0, 0))
    q, lk, lv, ck, cv = pl.pallas_call(
        _proj_kernel,
        grid=(NB,),
        in_specs=[
            pl.BlockSpec((TS, HM), lambda i: (i, 0)),
            full((HM, NH * HD)), full((1, NH * HD)),
            full((HM, 1)), full((1, 1)),
            full((HM, HD)), full((1, HD)),
            full((HM, HD)), full((1, HD)),
            full((HM, NH * HD)), full((1, NH * HD)),
            full((HM, NH * HD)), full((1, NH * HD)),
            full((1, HD)), full((1, HD)),
        ],
        out_specs=(
            pl.BlockSpec((TS, NH * HD), lambda i: (i, 0)),
            pl.BlockSpec((TS, NH * HD), lambda i: (i, 0)),
            pl.BlockSpec((TS, NH * HD), lambda i: (i, 0)),
            pl.BlockSpec((CB, HD), lambda i: (i, 0)),
            pl.BlockSpec((CB, HD), lambda i: (i, 0)),
        ),
        out_shape=(
            jax.ShapeDtypeStruct((S, NH * HD), jnp.float32),
            jax.ShapeDtypeStruct((S, NH * HD), jnp.float32),
            jax.ShapeDtypeStruct((S, NH * HD), jnp.float32),
            jax.ShapeDtypeStruct((C, HD), jnp.float32),
            jax.ShapeDtypeStruct((C, HD), jnp.float32),
        ),
    )(hs, Wq, bq2, Wcomp, bcomp2, Wk, bk2, Wv, bv2, Wlk, blk2, Wlv, blv2,
      gq2, gk2)

    merged = pl.pallas_call(
        _attn_kernel,
        grid=(NB,),
        in_specs=[
            pl.BlockSpec((TS, NH * HD), lambda i: (i, 0)),
            pl.BlockSpec((S, NH * HD), lambda i: (0, 0)),
            pl.BlockSpec((S, NH * HD), lambda i: (0, 0)),
            pl.BlockSpec((C, HD), lambda i: (0, 0)),
            pl.BlockSpec((C, HD), lambda i: (0, 0)),
        ],
        out_specs=pl.BlockSpec((TS, NH * HD), lambda i: (i, 0)),
        out_shape=jax.ShapeDtypeStruct((S, NH * HD), jnp.float32),
    )(q, lk, lv, ck, cv)

    out = pl.pallas_call(
        _out_kernel,
        grid=(NB,),
        in_specs=[
            pl.BlockSpec((TS, NH * HD), lambda i: (i, 0)),
            pl.BlockSpec((NH * HD, HM), lambda i: (0, 0)),
            pl.BlockSpec((1, HM), lambda i: (0, 0)),
        ],
        out_specs=pl.BlockSpec((TS, HM), lambda i: (i, 0)),
        out_shape=jax.ShapeDtypeStruct((S, HM), jnp.float32),
    )(merged, Wo, bo2)

    return out.reshape(B, S, HM)


# speedup vs baseline: 1.6802x; 1.0902x over previous
import numpy as np
import jax
import jax.numpy as jnp
from jax.experimental import pallas as pl

B, S, HM = 1, 2048, 1024
NH, HD = 16, 64
RATIO, TOPK, WINDOW = 4, 8, 128
ROPE_DIM = HD // 2
HALF = ROPE_DIM // 2
C = S // RATIO
TS = 256
CB = TS // RATIO
KW = TS + WINDOW
NB = S // TS
SCALE = 1.0 / np.sqrt(HD)
NEG = -1e9


def _rope_tables(pos_f):
    j = jax.lax.broadcasted_iota(jnp.int32, pos_f.shape, 1).astype(jnp.float32)
    freqs = jnp.exp(j * (-np.log(10000.0) / HALF))
    ang = pos_f * freqs
    return jnp.cos(ang), jnp.sin(ang)


def _norm_rope(x, g, cos, sin):
    ms = jnp.mean(x * x, axis=-1, keepdims=True)
    xn = x * jax.lax.rsqrt(ms + 1e-6) * g
    x1 = xn[:, :HALF]
    x2 = xn[:, HALF:ROPE_DIM]
    xp = xn[:, ROPE_DIM:]
    r1 = x1 * cos - x2 * sin
    r2 = x1 * sin + x2 * cos
    return jnp.concatenate([r1, r2, xp], axis=1)


def _proj_kernel(hs_ref, Wq_ref, bq_ref, Wcomp_ref, bcomp_ref, Wk_ref, bk_ref,
                 Wv_ref, bv_ref, Wlk_ref, blk_ref, Wlv_ref, blv_ref,
                 gq_ref, gk_ref,
                 q_ref, lk_ref, lv_ref, ck_ref, cv_ref):
    i = pl.program_id(0)
    hs = hs_ref[:]

    pos = (i * TS + jax.lax.broadcasted_iota(jnp.int32, (TS, HALF), 0)
           ).astype(jnp.float32)
    cos, sin = _rope_tables(pos)

    q = jnp.dot(hs, Wq_ref[:], preferred_element_type=jnp.float32) + bq_ref[:]
    lk = jnp.dot(hs, Wlk_ref[:], preferred_element_type=jnp.float32) + blk_ref[:]
    lv_ref[:] = jnp.dot(hs, Wlv_ref[:], preferred_element_type=jnp.float32) + blv_ref[:]
    gq = gq_ref[:]
    gk = gk_ref[:]
    for h in range(NH):
        sl = slice(h * HD, (h + 1) * HD)
        q_ref[:, sl] = _norm_rope(q[:, sl], gq, cos, sin)
        lk_ref[:, sl] = _norm_rope(lk[:, sl], gk, cos, sin)

    HIP = jax.lax.Precision.HIGHEST
    bf = lambda x: x.astype(jnp.bfloat16).astype(jnp.float32)
    sc = jnp.dot(hs, Wcomp_ref[:], preferred_element_type=jnp.float32) + bcomp_ref[:]
    e = jnp.exp(sc - jnp.max(sc))
    arow = jax.lax.broadcasted_iota(jnp.int32, (CB, TS), 0)
    acol = jax.lax.broadcasted_iota(jnp.int32, (CB, TS), 1)
    A = (acol // RATIO == arow).astype(jnp.float32)
    gs = jnp.dot(A, e, preferred_element_type=jnp.float32, precision=HIP)
    trow = jax.lax.broadcasted_iota(jnp.int32, (TS, CB), 0)
    tcol = jax.lax.broadcasted_iota(jnp.int32, (TS, CB), 1)
    At = (trow // RATIO == tcol).astype(jnp.float32)
    gsum_t = jnp.dot(At, gs, preferred_element_type=jnp.float32, precision=HIP)
    w = e / gsum_t
    entries = jnp.dot(A, bf(hs) * bf(w), preferred_element_type=jnp.float32,
                      precision=HIP)

    ck = jnp.dot(entries, Wk_ref[:], preferred_element_type=jnp.float32) + bk_ref[:]
    cv = jnp.dot(entries, Wv_ref[:], preferred_element_type=jnp.float32) + bv_ref[:]
    cpos = ((i * CB + jax.lax.broadcasted_iota(jnp.int32, (CB, HALF), 0))
            * RATIO + (RATIO - 1)).astype(jnp.float32)
    cosc, sinc = _rope_tables(cpos)
    ck_ref[:] = _norm_rope(ck, gk, cosc, sinc)
    cv_ref[:] = cv


def _attn_kernel(q_ref, lk_ref, lv_ref, ck_ref, cv_ref, o_ref):
    i = pl.program_id(0)

    ks = pl.multiple_of(jnp.maximum(i * TS - WINDOW, 0), WINDOW)
    kwin = lk_ref[pl.ds(ks, KW), :]
    vwin = lv_ref[pl.ds(ks, KW), :]
    qp = i * TS + jax.lax.broadcasted_iota(jnp.int32, (TS, KW), 0)
    kp = ks + jax.lax.broadcasted_iota(jnp.int32, (TS, KW), 1)
    diff = qp - kp
    vis_l = (diff >= 0) & (diff <= WINDOW)

    def body(CW):
        ck = ck_ref[:CW, :]
        cv = cv_ref[:CW, :]
        qpos_c = i * TS + jax.lax.broadcasted_iota(jnp.int32, (TS, CW), 0)
        cstart = RATIO * jax.lax.broadcasted_iota(jnp.int32, (TS, CW), 1)
        vis_c = cstart <= qpos_c

        for h in range(NH):
            hs_ = slice(h * HD, (h + 1) * HD)
            qb = q_ref[:, hs_]

            s = jax.lax.dot_general(qb, ck, (((1,), (1,)), ((), ())),
                                    preferred_element_type=jnp.float32) * SCALE
            s = jnp.where(vis_c, s, NEG)

            work = s
            m0 = None
            for _ in range(TOPK):
                mk = jnp.max(work, axis=1, keepdims=True)
                if m0 is None:
                    m0 = mk
                thresh = mk
                work = jnp.where(work >= mk, -jnp.inf, work)
            P = jnp.where(s >= thresh, jnp.exp(s - m0), 0.0)
            norm = jnp.sum(P, axis=1, keepdims=True)
            ctx_s = jnp.dot(P, cv, preferred_element_type=jnp.float32) / norm

            sl = jax.lax.dot_general(qb, kwin[:, hs_], (((1,), (1,)), ((), ())),
                                     preferred_element_type=jnp.float32) * SCALE
            sl = jnp.where(vis_l, sl, NEG)
            lm = jnp.max(sl, axis=1, keepdims=True)
            le = jnp.exp(sl - lm)
            ctx_l = (jnp.dot(le, vwin[:, hs_], preferred_element_type=jnp.float32)
                     / jnp.sum(le, axis=1, keepdims=True))

            o_ref[:, hs_] = (ctx_s + ctx_l) * 0.5

    body(C)


def _out_kernel(m_ref, Wo_ref, bo_ref, o_ref):
    o_ref[:] = (jnp.dot(m_ref[:], Wo_ref[:], preferred_element_type=jnp.float32)
                + bo_ref[:])


def kernel(hidden_states, Wq, bq, Wcomp, bcomp, Wk, bk, Wv, bv,
           Wlk, blk, Wlv, blv, gq, gk, Wo, bo):
    hs = hidden_states.reshape(S, HM)
    bq2 = bq.reshape(1, -1)
    bcomp2 = bcomp.reshape(1, 1)
    bk2 = bk.reshape(1, -1)
    bv2 = bv.reshape(1, -1)
    blk2 = blk.reshape(1, -1)
    blv2 = blv.reshape(1, -1)
    gq2 = gq.reshape(1, -1)
    gk2 = gk.reshape(1, -1)
    bo2 = bo.reshape(1, -1)

    full = lambda shape: pl.BlockSpec(shape, lambda i: (0, 0))
    q, lk, lv, ck, cv = pl.pallas_call(
        _proj_kernel,
        grid=(NB,),
        in_specs=[
            pl.BlockSpec((TS, HM), lambda i: (i, 0)),
            full((HM, NH * HD)), full((1, NH * HD)),
            full((HM, 1)), full((1, 1)),
            full((HM, HD)), full((1, HD)),
            full((HM, HD)), full((1, HD)),
            full((HM, NH * HD)), full((1, NH * HD)),
            full((HM, NH * HD)), full((1, NH * HD)),
            full((1, HD)), full((1, HD)),
        ],
        out_specs=(
            pl.BlockSpec((TS, NH * HD), lambda i: (i, 0)),
            pl.BlockSpec((TS, NH * HD), lambda i: (i, 0)),
            pl.BlockSpec((TS, NH * HD), lambda i: (i, 0)),
            pl.BlockSpec((CB, HD), lambda i: (i, 0)),
            pl.BlockSpec((CB, HD), lambda i: (i, 0)),
        ),
        out_shape=(
            jax.ShapeDtypeStruct((S, NH * HD), jnp.float32),
            jax.ShapeDtypeStruct((S, NH * HD), jnp.float32),
            jax.ShapeDtypeStruct((S, NH * HD), jnp.float32),
            jax.ShapeDtypeStruct((C, HD), jnp.float32),
            jax.ShapeDtypeStruct((C, HD), jnp.float32),
        ),
    )(hs, Wq, bq2, Wcomp, bcomp2, Wk, bk2, Wv, bv2, Wlk, blk2, Wlv, blv2,
      gq2, gk2)

    merged = pl.pallas_call(
        _attn_kernel,
        grid=(NB,),
        in_specs=[
            pl.BlockSpec((TS, NH * HD), lambda i: (i, 0)),
            pl.BlockSpec((S, NH * HD), lambda i: (0, 0)),
            pl.BlockSpec((S, NH * HD), lambda i: (0, 0)),
            pl.BlockSpec((C, HD), lambda i: (0, 0)),
            pl.BlockSpec((C, HD), lambda i: (0, 0)),
        ],
        out_specs=pl.BlockSpec((TS, NH * HD), lambda i: (i, 0)),
        out_shape=jax.ShapeDtypeStruct((S, NH * HD), jnp.float32),
    )(q, lk, lv, ck, cv)

    out = pl.pallas_call(
        _out_kernel,
        grid=(NB,),
        in_specs=[
            pl.BlockSpec((TS, NH * HD), lambda i: (i, 0)),
            pl.BlockSpec((NH * HD, HM), lambda i: (0, 0)),
            pl.BlockSpec((1, HM), lambda i: (0, 0)),
        ],
        out_specs=pl.BlockSpec((TS, HM), lambda i: (i, 0)),
        out_shape=jax.ShapeDtypeStruct((S, HM), jnp.float32),
    )(merged, Wo, bo2)

    return out.reshape(B, S, HM)
